# baseline (speedup 1.0000x reference)
import functools
import math

import jax
import jax.numpy as jnp
from jax import lax
from jax.experimental import pallas as pl
from jax.experimental.pallas import tpu as pltpu
from jax.experimental.pallas import tpu_sc as plsc

D = 768
H = 12
HD = 64
MLP = 3072
E = 64
GS = 1024
CAPF = 1.05
CAP = int(math.ceil(CAPF * GS / E))
G = 4
T = 4096
SLOT = 72
NSLOT = E * SLOT
QT = 256

F32 = jnp.float32
BF16 = jnp.bfloat16

_SC_NC = 2
_SC_NS = 16
_SC_NW = _SC_NC * _SC_NS


def _attn_body(h_ref, lns_ref, lnb_ref, wq_ref, bq_ref, wk_ref, bk_ref,
               wv_ref, bv_ref, wo_ref, bo_ref, out_ref,
               qb_ref, kb_ref, vse_ref):
    qt = pl.program_id(1)
    S = h_ref.shape[1]

    @pl.when(qt == 0)
    def _():
        xf = h_ref[0]
        m = jnp.mean(xf, axis=1, keepdims=True)
        xc = xf - m
        va = jnp.mean(xc * xc, axis=1, keepdims=True)
        xb = (xc * lax.rsqrt(va + 1e-6) * lns_ref[...]
              + lnb_ref[...]).astype(BF16)
        qb_ref[...] = ((jnp.dot(xb, wq_ref[...], preferred_element_type=F32)
                        + bq_ref[...]) * (1.0 / math.sqrt(HD))).astype(BF16)
        kb_ref[...] = (jnp.dot(xb, wk_ref[...], preferred_element_type=F32)
                       + bk_ref[...]).astype(BF16)
        v = (jnp.dot(xb, wv_ref[...], preferred_element_type=F32)
             + bv_ref[...])
        ones = jnp.ones((S, 1), F32)
        for hh in range(H):
            vse_ref[hh] = jnp.concatenate(
                [v[:, hh * HD:(hh + 1) * HD], ones], axis=1)

    rows = pl.ds(qt * QT, QT)
    o_list = []
    for hh in range(H):
        hcols = slice(hh * HD, (hh + 1) * HD)
        qs = qb_ref[rows, hcols]
        ks = kb_ref[:, hcols]
        s = lax.dot_general(qs, ks, (((1,), (1,)), ((), ())),
                            preferred_element_type=F32)
        e = jnp.exp(s - jnp.max(s, axis=1, keepdims=True))
        oext = jnp.dot(e, vse_ref[hh], preferred_element_type=F32)
        r = 1.0 / oext[:, HD:HD + 1]
        o_list.append((oext[:, :HD] * r).astype(BF16))
    oa = jnp.concatenate(o_list, axis=1)
    out_ref[0] = (h_ref[0, rows, :] + bo_ref[...]
                  + jnp.dot(oa, wo_ref[...], preferred_element_type=F32))


def _attn_block(h, blk):
    b, s, _ = h.shape
    wq = blk['wq'].reshape(D, D).astype(BF16)
    wk = blk['wk'].reshape(D, D).astype(BF16)
    wv = blk['wv'].reshape(D, D).astype(BF16)
    wo = blk['wo'].reshape(D, D).astype(BF16)
    bq = blk['bq'].reshape(1, D)
    bk = blk['bk'].reshape(1, D)
    bv = blk['bv'].reshape(1, D)
    bo = blk['bo'].reshape(1, D)
    lns = blk['ln1_s'].reshape(1, D)
    lnb = blk['ln1_b'].reshape(1, D)
    return pl.pallas_call(
        _attn_body,
        grid=(b, s // QT),
        in_specs=[
            pl.BlockSpec((1, s, D), lambda i, j: (i, 0, 0)),
            pl.BlockSpec((1, D), lambda i, j: (0, 0)),
            pl.BlockSpec((1, D), lambda i, j: (0, 0)),
            pl.BlockSpec((D, D), lambda i, j: (0, 0)),
            pl.BlockSpec((1, D), lambda i, j: (0, 0)),
            pl.BlockSpec((D, D), lambda i, j: (0, 0)),
            pl.BlockSpec((1, D), lambda i, j: (0, 0)),
            pl.BlockSpec((D, D), lambda i, j: (0, 0)),
            pl.BlockSpec((1, D), lambda i, j: (0, 0)),
            pl.BlockSpec((D, D), lambda i, j: (0, 0)),
            pl.BlockSpec((1, D), lambda i, j: (0, 0)),
        ],
        out_specs=pl.BlockSpec((1, QT, D), lambda i, j: (i, j, 0)),
        out_shape=jax.ShapeDtypeStruct((b, s, D), F32),
        compiler_params=pltpu.CompilerParams(
            dimension_semantics=("parallel", "arbitrary")),
        scratch_shapes=[pltpu.VMEM((s, D), BF16), pltpu.VMEM((s, D), BF16),
                        pltpu.VMEM((H, s, HD + 1), F32)],
    )(h, lns, lnb, wq, bq, wk, bk, wv, bv, wo, bo)


def _mlp_body(x_ref, lns_ref, lnb_ref, w1_ref, b1_ref, w2_ref, b2_ref,
              out_ref):
    xf = x_ref[...]
    m = jnp.mean(xf, axis=1, keepdims=True)
    xc = xf - m
    v = jnp.mean(xc * xc, axis=1, keepdims=True)
    xln = xc * lax.rsqrt(v + 1e-6) * lns_ref[...] + lnb_ref[...]
    h1 = jax.nn.gelu(jnp.dot(xln.astype(BF16), w1_ref[...],
                             preferred_element_type=F32) + b1_ref[...])
    y = jnp.dot(h1.astype(BF16), w2_ref[...],
                preferred_element_type=F32) + b2_ref[...]
    out_ref[...] = xf + y


def _mlp_block(h, blk):
    b, s, _ = h.shape
    x = h.reshape(T, D)
    out = pl.pallas_call(
        _mlp_body,
        grid=(T // 512,),
        in_specs=[
            pl.BlockSpec((512, D), lambda i: (i, 0)),
            pl.BlockSpec((1, D), lambda i: (0, 0)),
            pl.BlockSpec((1, D), lambda i: (0, 0)),
            pl.BlockSpec((D, MLP), lambda i: (0, 0)),
            pl.BlockSpec((1, MLP), lambda i: (0, 0)),
            pl.BlockSpec((MLP, D), lambda i: (0, 0)),
            pl.BlockSpec((1, D), lambda i: (0, 0)),
        ],
        out_specs=pl.BlockSpec((512, D), lambda i: (i, 0)),
        out_shape=jax.ShapeDtypeStruct((T, D), F32),
        compiler_params=pltpu.CompilerParams(
            dimension_semantics=("parallel",)),
    )(x, blk['ln2_s'].reshape(1, D), blk['ln2_b'].reshape(1, D),
      blk['w1'].astype(BF16), blk['b1'].reshape(1, MLP),
      blk['w2'].astype(BF16), blk['b2'].reshape(1, D))
    return out.reshape(b, s, D)


def _router_body(x_ref, lns_ref, lnb_ref, wr_ref, xln_ref, st_ref, gate_ref,
                 slot_ref, aux_ref):
    g = pl.program_id(0)
    xf = x_ref[0]
    m = jnp.mean(xf, axis=1, keepdims=True)
    xc = xf - m
    v = jnp.mean(xc * xc, axis=1, keepdims=True)
    xln = xc * lax.rsqrt(v + 1e-6) * lns_ref[...] + lnb_ref[...]
    xln_ref[0] = xln

    logits = jnp.dot(xln, wr_ref[...], preferred_element_type=F32)
    mx = jnp.max(logits, axis=1, keepdims=True)
    ex = jnp.exp(logits - mx)
    gates = ex / jnp.sum(ex, axis=1, keepdims=True)

    gv = jnp.max(gates, axis=1)
    cols = lax.broadcasted_iota(jnp.int32, (GS, E), 1)
    gi = jnp.min(jnp.where(gates == gv[:, None], cols, E), axis=1)
    moh = (cols == gi[:, None]).astype(F32)

    ri = lax.broadcasted_iota(jnp.int32, (GS, GS), 0)
    ci = lax.broadcasted_iota(jnp.int32, (GS, GS), 1)
    tri = (ri >= ci).astype(BF16)
    pos = lax.dot_general(tri, moh.astype(BF16), (((1,), (0,)), ((), ())),
                          preferred_element_type=F32) - 1.0
    postok = jnp.sum(pos * moh, axis=1)
    keep = (postok < CAP).astype(F32)
    gate_ref[0, 0] = gv * keep
    slotf = (gi.astype(F32) * SLOT + g.astype(F32) * CAP + postok) * keep
    slot_ref[0, 0] = slotf.astype(jnp.int32)

    colt = lax.broadcasted_iota(jnp.int32, (GS, E * CAP), 1)
    e_col = colt // CAP
    c_col = colt % CAP
    posi = postok.astype(jnp.int32)
    oh = ((gi[:, None] == e_col) & (posi[:, None] == c_col)
          & (keep[:, None] > 0)).astype(F32)
    trow = (lax.broadcasted_iota(jnp.int32, (GS, E * CAP), 0).astype(F32)
            + g.astype(F32) * GS)
    st = jnp.sum(oh * trow, axis=0)
    st_ref[0, 0] = st.astype(jnp.int32)

    imp = jnp.sum(gates, axis=0)
    load = jnp.sum(moh, axis=0)
    im = jnp.mean(imp)
    iv = jnp.mean((imp - im) ** 2)
    lm = jnp.mean(load)
    lv = jnp.mean((load - lm) ** 2)
    part = (0.5 / G) * (iv / (im * im + 1e-10) + lv / (lm * lm + 1e-10))
    aux_ref[...] = jnp.reshape(part, (1, 1, 1))


def _router(xg, blk):
    return pl.pallas_call(
        _router_body,
        grid=(G,),
        in_specs=[
            pl.BlockSpec((1, GS, D), lambda g: (g, 0, 0)),
            pl.BlockSpec((1, D), lambda g: (0, 0)),
            pl.BlockSpec((1, D), lambda g: (0, 0)),
            pl.BlockSpec((D, E), lambda g: (0, 0)),
        ],
        out_specs=[
            pl.BlockSpec((1, GS, D), lambda g: (g, 0, 0)),
            pl.BlockSpec((1, 1, E * CAP), lambda g: (g, 0, 0)),
            pl.BlockSpec((1, 1, GS), lambda g: (g, 0, 0)),
            pl.BlockSpec((1, 1, GS), lambda g: (g, 0, 0)),
            pl.BlockSpec((1, 1, 1), lambda g: (g, 0, 0)),
        ],
        out_shape=[
            jax.ShapeDtypeStruct((G, GS, D), F32),
            jax.ShapeDtypeStruct((G, 1, E * CAP), jnp.int32),
            jax.ShapeDtypeStruct((G, 1, GS), F32),
            jax.ShapeDtypeStruct((G, 1, GS), jnp.int32),
            jax.ShapeDtypeStruct((G, 1, 1), F32),
        ],
        compiler_params=pltpu.CompilerParams(
            dimension_semantics=("parallel",)),
    )(xg, blk['ln2_s'].reshape(1, D), blk['ln2_b'].reshape(1, D), blk['wr'])


def _expert_body(x_ref, w1_ref, b1_ref, w2_ref, b2_ref, out_ref):
    f = pl.program_id(1)
    xb = x_ref[0].astype(BF16)
    h1 = jax.nn.gelu(jnp.dot(xb, w1_ref[0].astype(BF16),
                             preferred_element_type=F32) + b1_ref[0])
    yc = jnp.dot(h1.astype(BF16), w2_ref[0].astype(BF16),
                 preferred_element_type=F32)

    @pl.when(f == 0)
    def _():
        out_ref[0] = b2_ref[0] + yc

    @pl.when(f > 0)
    def _():
        out_ref[0] += yc


def _expert_ffn(buf, blk):
    FT = 768
    return pl.pallas_call(
        _expert_body,
        grid=(E, MLP // FT),
        in_specs=[
            pl.BlockSpec((1, SLOT, D), lambda e, f: (e, 0, 0)),
            pl.BlockSpec((1, D, FT), lambda e, f: (e, 0, f)),
            pl.BlockSpec((1, 1, FT), lambda e, f: (e, 0, f)),
            pl.BlockSpec((1, FT, D), lambda e, f: (e, f, 0)),
            pl.BlockSpec((1, 1, D), lambda e, f: (e, 0, 0)),
        ],
        out_specs=pl.BlockSpec((1, SLOT, D), lambda e, f: (e, 0, 0)),
        out_shape=jax.ShapeDtypeStruct((E, SLOT, D), F32),
        compiler_params=pltpu.CompilerParams(
            dimension_semantics=("parallel", "arbitrary")),
    )(buf, blk['ew1'], blk['eb1'].reshape(E, 1, MLP),
      blk['ew2'], blk['eb2'].reshape(E, 1, D))


def _final_body(h_ref, y_ref, g_ref, lns_ref, lnb_ref, out_ref):
    xf = h_ref[...] + g_ref[...] * y_ref[...]
    m = jnp.mean(xf, axis=1, keepdims=True)
    xc = xf - m
    v = jnp.mean(xc * xc, axis=1, keepdims=True)
    out_ref[...] = xc * lax.rsqrt(v + 1e-6) * lns_ref[...] + lnb_ref[...]


def _final(h, y, gate, lns, lnb):
    return pl.pallas_call(
        _final_body,
        grid=(T // 512,),
        in_specs=[
            pl.BlockSpec((512, D), lambda i: (i, 0)),
            pl.BlockSpec((512, D), lambda i: (i, 0)),
            pl.BlockSpec((512, 1), lambda i: (i, 0)),
            pl.BlockSpec((1, D), lambda i: (0, 0)),
            pl.BlockSpec((1, D), lambda i: (0, 0)),
        ],
        out_specs=pl.BlockSpec((512, D), lambda i: (i, 0)),
        out_shape=jax.ShapeDtypeStruct((T, D), F32),
        compiler_params=pltpu.CompilerParams(
            dimension_semantics=("parallel",)),
    )(h, y, gate, lns.reshape(1, D), lnb.reshape(1, D))


def _sc_gather(table, idx):
    n = idx.shape[0]
    d = table.shape[1]
    sub = d // 128
    table2 = table.reshape(table.shape[0] * sub, 128)
    idx2 = (idx[:, None] * sub
            + jnp.arange(sub, dtype=jnp.int32)[None, :]).reshape(-1)
    n2 = n * sub
    b_per_w = n2 // _SC_NW
    NWIN = 4
    W = b_per_w // NWIN
    mesh = plsc.VectorSubcoreMesh(core_axis_name="c", subcore_axis_name="s")

    @functools.partial(
        pl.kernel, mesh=mesh,
        out_type=jax.ShapeDtypeStruct((n2, 128), F32),
        scratch_types=(
            [pltpu.VMEM((b_per_w,), jnp.int32)]
            + [pltpu.VMEM((W, 128), F32)] * NWIN
            + [pltpu.SemaphoreType.DMA] * NWIN
        ),
    )
    def k(table_hbm, idx_hbm, out_hbm, idx_v, *bufsem):
        bufs = bufsem[:NWIN]
        sems = bufsem[NWIN:]
        wid = lax.axis_index("s") * _SC_NC + lax.axis_index("c")
        base = wid * b_per_w
        pltpu.sync_copy(idx_hbm.at[pl.ds(base, b_per_w)], idx_v)
        cps = [pltpu.async_copy(table_hbm.at[idx_v.at[pl.ds(j * W, W)]],
                                bufs[j], sems[j]) for j in range(NWIN)]
        for j in range(NWIN):
            cps[j].wait()
            pltpu.sync_copy(bufs[j], out_hbm.at[pl.ds(base + j * W, W)])

    return k(table2, idx2).reshape(n, d)


def kernel(x, params):
    p = params
    b, s, _ = x.shape
    h = x + p['posemb']
    h = _attn_block(h, p['block0'])
    h = _mlp_block(h, p['block0'])
    h = _attn_block(h, p['block1'])

    blk = p['block1']
    xg = h.reshape(G, GS, D)
    xln, st, gate, tokslot, aux = _router(xg, blk)

    disp_idx = (st.reshape(G, E, CAP).transpose(1, 0, 2).reshape(E, G * CAP))
    disp_idx = jnp.pad(disp_idx, ((0, 0), (0, SLOT - G * CAP))).reshape(NSLOT)

    buf = _sc_gather(xln.reshape(T, D), disp_idx)
    ebuf = _expert_ffn(buf.reshape(E, SLOT, D), blk)
    yraw = _sc_gather(ebuf.reshape(NSLOT, D), tokslot.reshape(T))

    out = _final(h.reshape(T, D), yraw, gate.reshape(T, 1),
                 p['lnf_s'], p['lnf_b'])
    return out.reshape(b, s, D), jnp.sum(aux)


# speedup vs baseline: 1.1066x; 1.1066x over previous
import functools
import math

import jax
import jax.numpy as jnp
from jax import lax
from jax.experimental import pallas as pl
from jax.experimental.pallas import tpu as pltpu
from jax.experimental.pallas import tpu_sc as plsc

D = 768
H = 12
HD = 64
MLP = 3072
E = 64
GS = 1024
CAPF = 1.05
CAP = int(math.ceil(CAPF * GS / E))
G = 4
T = 4096
SLOT = 72
NSLOT = E * SLOT
QT = 256

F32 = jnp.float32
BF16 = jnp.bfloat16

_SC_NC = 2
_SC_NS = 16
_SC_NW = _SC_NC * _SC_NS


def _attn_body(h_ref, lns_ref, lnb_ref, wq_ref, bq_ref, wk_ref, bk_ref,
               wv_ref, bv_ref, wo_ref, bo_ref, out_ref,
               qb_ref, kb_ref, vse_ref):
    qt = pl.program_id(1)
    S = h_ref.shape[1]

    @pl.when(qt == 0)
    def _():
        xf = h_ref[0]
        m = jnp.mean(xf, axis=1, keepdims=True)
        xc = xf - m
        va = jnp.mean(xc * xc, axis=1, keepdims=True)
        xb = (xc * lax.rsqrt(va + 1e-6) * lns_ref[...]
              + lnb_ref[...]).astype(BF16)
        qb_ref[...] = ((jnp.dot(xb, wq_ref[...], preferred_element_type=F32)
                        + bq_ref[...]) * (1.0 / math.sqrt(HD))).astype(BF16)
        kb_ref[...] = (jnp.dot(xb, wk_ref[...], preferred_element_type=F32)
                       + bk_ref[...]).astype(BF16)
        v = (jnp.dot(xb, wv_ref[...], preferred_element_type=F32)
             + bv_ref[...])
        ones = jnp.ones((S, 1), F32)
        for hh in range(H):
            vse_ref[hh] = jnp.concatenate(
                [v[:, hh * HD:(hh + 1) * HD], ones], axis=1)

    rows = pl.ds(qt * QT, QT)
    o_list = []
    for hh in range(H):
        hcols = slice(hh * HD, (hh + 1) * HD)
        qs = qb_ref[rows, hcols]
        ks = kb_ref[:, hcols]
        s = lax.dot_general(qs, ks, (((1,), (1,)), ((), ())),
                            preferred_element_type=F32)
        e = jnp.exp(s - jnp.max(s, axis=1, keepdims=True))
        oext = jnp.dot(e, vse_ref[hh], preferred_element_type=F32)
        r = 1.0 / oext[:, HD:HD + 1]
        o_list.append((oext[:, :HD] * r).astype(BF16))
    oa = jnp.concatenate(o_list, axis=1)
    out_ref[0] = (h_ref[0, rows, :] + bo_ref[...]
                  + jnp.dot(oa, wo_ref[...], preferred_element_type=F32))


def _attn_block(h, blk):
    b, s, _ = h.shape
    wq = blk['wq'].reshape(D, D).astype(BF16)
    wk = blk['wk'].reshape(D, D).astype(BF16)
    wv = blk['wv'].reshape(D, D).astype(BF16)
    wo = blk['wo'].reshape(D, D).astype(BF16)
    bq = blk['bq'].reshape(1, D)
    bk = blk['bk'].reshape(1, D)
    bv = blk['bv'].reshape(1, D)
    bo = blk['bo'].reshape(1, D)
    lns = blk['ln1_s'].reshape(1, D)
    lnb = blk['ln1_b'].reshape(1, D)
    return pl.pallas_call(
        _attn_body,
        grid=(b, s // QT),
        in_specs=[
            pl.BlockSpec((1, s, D), lambda i, j: (i, 0, 0)),
            pl.BlockSpec((1, D), lambda i, j: (0, 0)),
            pl.BlockSpec((1, D), lambda i, j: (0, 0)),
            pl.BlockSpec((D, D), lambda i, j: (0, 0)),
            pl.BlockSpec((1, D), lambda i, j: (0, 0)),
            pl.BlockSpec((D, D), lambda i, j: (0, 0)),
            pl.BlockSpec((1, D), lambda i, j: (0, 0)),
            pl.BlockSpec((D, D), lambda i, j: (0, 0)),
            pl.BlockSpec((1, D), lambda i, j: (0, 0)),
            pl.BlockSpec((D, D), lambda i, j: (0, 0)),
            pl.BlockSpec((1, D), lambda i, j: (0, 0)),
        ],
        out_specs=pl.BlockSpec((1, QT, D), lambda i, j: (i, j, 0)),
        out_shape=jax.ShapeDtypeStruct((b, s, D), F32),
        compiler_params=pltpu.CompilerParams(
            dimension_semantics=("parallel", "arbitrary")),
        scratch_shapes=[pltpu.VMEM((s, D), BF16), pltpu.VMEM((s, D), BF16),
                        pltpu.VMEM((H, s, HD + 1), F32)],
    )(h, lns, lnb, wq, bq, wk, bk, wv, bv, wo, bo)


def _mlp_body(x_ref, lns_ref, lnb_ref, w1_ref, b1_ref, w2_ref, b2_ref,
              out_ref):
    xf = x_ref[...]
    m = jnp.mean(xf, axis=1, keepdims=True)
    xc = xf - m
    v = jnp.mean(xc * xc, axis=1, keepdims=True)
    xln = xc * lax.rsqrt(v + 1e-6) * lns_ref[...] + lnb_ref[...]
    h1 = jax.nn.gelu(jnp.dot(xln.astype(BF16), w1_ref[...],
                             preferred_element_type=F32) + b1_ref[...])
    y = jnp.dot(h1.astype(BF16), w2_ref[...],
                preferred_element_type=F32) + b2_ref[...]
    out_ref[...] = xf + y


def _mlp_block(h, blk):
    b, s, _ = h.shape
    x = h.reshape(T, D)
    out = pl.pallas_call(
        _mlp_body,
        grid=(T // 512,),
        in_specs=[
            pl.BlockSpec((512, D), lambda i: (i, 0)),
            pl.BlockSpec((1, D), lambda i: (0, 0)),
            pl.BlockSpec((1, D), lambda i: (0, 0)),
            pl.BlockSpec((D, MLP), lambda i: (0, 0)),
            pl.BlockSpec((1, MLP), lambda i: (0, 0)),
            pl.BlockSpec((MLP, D), lambda i: (0, 0)),
            pl.BlockSpec((1, D), lambda i: (0, 0)),
        ],
        out_specs=pl.BlockSpec((512, D), lambda i: (i, 0)),
        out_shape=jax.ShapeDtypeStruct((T, D), F32),
        compiler_params=pltpu.CompilerParams(
            dimension_semantics=("parallel",)),
    )(x, blk['ln2_s'].reshape(1, D), blk['ln2_b'].reshape(1, D),
      blk['w1'].astype(BF16), blk['b1'].reshape(1, MLP),
      blk['w2'].astype(BF16), blk['b2'].reshape(1, D))
    return out.reshape(b, s, D)


def _router_body(x_ref, lns_ref, lnb_ref, wr_ref, xln_ref, st_ref, gate_ref,
                 slot_ref, aux_ref):
    g = pl.program_id(0)
    xf = x_ref[0]
    m = jnp.mean(xf, axis=1, keepdims=True)
    xc = xf - m
    v = jnp.mean(xc * xc, axis=1, keepdims=True)
    xln = xc * lax.rsqrt(v + 1e-6) * lns_ref[...] + lnb_ref[...]
    xln_ref[0] = xln

    logits = jnp.dot(xln, wr_ref[...], preferred_element_type=F32)
    mx = jnp.max(logits, axis=1, keepdims=True)
    ex = jnp.exp(logits - mx)
    gates = ex / jnp.sum(ex, axis=1, keepdims=True)

    gv = jnp.max(gates, axis=1)
    cols = lax.broadcasted_iota(jnp.int32, (GS, E), 1)
    gi = jnp.min(jnp.where(gates == gv[:, None], cols, E), axis=1)
    moh = (cols == gi[:, None]).astype(F32)

    ri = lax.broadcasted_iota(jnp.int32, (GS, GS), 0)
    ci = lax.broadcasted_iota(jnp.int32, (GS, GS), 1)
    tri = (ri >= ci).astype(BF16)
    pos = lax.dot_general(tri, moh.astype(BF16), (((1,), (0,)), ((), ())),
                          preferred_element_type=F32) - 1.0
    postok = jnp.sum(pos * moh, axis=1)
    keep = (postok < CAP).astype(F32)
    gate_ref[0, 0] = gv * keep
    slotf = (gi.astype(F32) * SLOT + g.astype(F32) * CAP + postok) * keep
    slot_ref[0, 0] = slotf.astype(jnp.int32)

    colt = lax.broadcasted_iota(jnp.int32, (GS, E * CAP), 1)
    e_col = colt // CAP
    c_col = colt % CAP
    posi = postok.astype(jnp.int32)
    oh = ((gi[:, None] == e_col) & (posi[:, None] == c_col)
          & (keep[:, None] > 0)).astype(F32)
    trow = (lax.broadcasted_iota(jnp.int32, (GS, E * CAP), 0).astype(F32)
            + g.astype(F32) * GS)
    st = jnp.sum(oh * trow, axis=0)
    st_ref[0, 0] = st.astype(jnp.int32)

    imp = jnp.sum(gates, axis=0)
    load = jnp.sum(moh, axis=0)
    im = jnp.mean(imp)
    iv = jnp.mean((imp - im) ** 2)
    lm = jnp.mean(load)
    lv = jnp.mean((load - lm) ** 2)
    part = (0.5 / G) * (iv / (im * im + 1e-10) + lv / (lm * lm + 1e-10))
    aux_ref[...] = jnp.reshape(part, (1, 1, 1))


def _router(xg, blk):
    return pl.pallas_call(
        _router_body,
        grid=(G,),
        in_specs=[
            pl.BlockSpec((1, GS, D), lambda g: (g, 0, 0)),
            pl.BlockSpec((1, D), lambda g: (0, 0)),
            pl.BlockSpec((1, D), lambda g: (0, 0)),
            pl.BlockSpec((D, E), lambda g: (0, 0)),
        ],
        out_specs=[
            pl.BlockSpec((1, GS, D), lambda g: (g, 0, 0)),
            pl.BlockSpec((1, 1, E * CAP), lambda g: (g, 0, 0)),
            pl.BlockSpec((1, 1, GS), lambda g: (g, 0, 0)),
            pl.BlockSpec((1, 1, GS), lambda g: (g, 0, 0)),
            pl.BlockSpec((1, 1, 1), lambda g: (g, 0, 0)),
        ],
        out_shape=[
            jax.ShapeDtypeStruct((G, GS, D), F32),
            jax.ShapeDtypeStruct((G, 1, E * CAP), jnp.int32),
            jax.ShapeDtypeStruct((G, 1, GS), F32),
            jax.ShapeDtypeStruct((G, 1, GS), jnp.int32),
            jax.ShapeDtypeStruct((G, 1, 1), F32),
        ],
        compiler_params=pltpu.CompilerParams(
            dimension_semantics=("parallel",)),
    )(xg, blk['ln2_s'].reshape(1, D), blk['ln2_b'].reshape(1, D), blk['wr'])


def _expert_body(x_ref, w1_ref, b1_ref, w2_ref, b2_ref, out_ref):
    xb = x_ref[0].astype(BF16)
    h1 = jax.nn.gelu(jnp.dot(xb, w1_ref[0].astype(BF16),
                             preferred_element_type=F32) + b1_ref[0])
    yc = jnp.dot(h1.astype(BF16), w2_ref[0].astype(BF16),
                 preferred_element_type=F32)
    out_ref[0] = b2_ref[0] + yc


def _expert_ffn(buf, blk):
    return pl.pallas_call(
        _expert_body,
        grid=(E,),
        in_specs=[
            pl.BlockSpec((1, SLOT, D), lambda e: (e, 0, 0)),
            pl.BlockSpec((1, D, MLP), lambda e: (e, 0, 0)),
            pl.BlockSpec((1, 1, MLP), lambda e: (e, 0, 0)),
            pl.BlockSpec((1, MLP, D), lambda e: (e, 0, 0)),
            pl.BlockSpec((1, 1, D), lambda e: (e, 0, 0)),
        ],
        out_specs=pl.BlockSpec((1, SLOT, D), lambda e: (e, 0, 0)),
        out_shape=jax.ShapeDtypeStruct((E, SLOT, D), F32),
        compiler_params=pltpu.CompilerParams(
            dimension_semantics=("parallel",)),
    )(buf, blk['ew1'], blk['eb1'].reshape(E, 1, MLP),
      blk['ew2'], blk['eb2'].reshape(E, 1, D))


def _final_body(h_ref, y_ref, g_ref, lns_ref, lnb_ref, out_ref):
    xf = h_ref[...] + g_ref[...] * y_ref[...]
    m = jnp.mean(xf, axis=1, keepdims=True)
    xc = xf - m
    v = jnp.mean(xc * xc, axis=1, keepdims=True)
    out_ref[...] = xc * lax.rsqrt(v + 1e-6) * lns_ref[...] + lnb_ref[...]


def _final(h, y, gate, lns, lnb):
    return pl.pallas_call(
        _final_body,
        grid=(T // 512,),
        in_specs=[
            pl.BlockSpec((512, D), lambda i: (i, 0)),
            pl.BlockSpec((512, D), lambda i: (i, 0)),
            pl.BlockSpec((512, 1), lambda i: (i, 0)),
            pl.BlockSpec((1, D), lambda i: (0, 0)),
            pl.BlockSpec((1, D), lambda i: (0, 0)),
        ],
        out_specs=pl.BlockSpec((512, D), lambda i: (i, 0)),
        out_shape=jax.ShapeDtypeStruct((T, D), F32),
        compiler_params=pltpu.CompilerParams(
            dimension_semantics=("parallel",)),
    )(h, y, gate, lns.reshape(1, D), lnb.reshape(1, D))


def _sc_gather(table, idx):
    n = idx.shape[0]
    d = table.shape[1]
    b_per_w = n // _SC_NW
    mesh = plsc.VectorSubcoreMesh(core_axis_name="c", subcore_axis_name="s")

    @functools.partial(
        pl.kernel, mesh=mesh,
        out_type=jax.ShapeDtypeStruct((n, d), F32),
        scratch_types=[
            pltpu.VMEM((b_per_w,), jnp.int32),
            pltpu.VMEM((b_per_w // 2, d), F32),
            pltpu.VMEM((b_per_w // 2, d), F32),
            pltpu.SemaphoreType.DMA,
            pltpu.SemaphoreType.DMA,
        ],
    )
    def k(table_hbm, idx_hbm, out_hbm, idx_v, buf0, buf1, sem0, sem1):
        W = b_per_w // 2
        wid = lax.axis_index("s") * _SC_NC + lax.axis_index("c")
        base = wid * b_per_w
        pltpu.sync_copy(idx_hbm.at[pl.ds(base, b_per_w)], idx_v)
        cp0 = pltpu.async_copy(table_hbm.at[idx_v.at[pl.ds(0, W)]], buf0, sem0)
        cp1 = pltpu.async_copy(table_hbm.at[idx_v.at[pl.ds(W, W)]], buf1, sem1)
        cp0.wait()
        pltpu.sync_copy(buf0, out_hbm.at[pl.ds(base, W)])
        cp1.wait()
        pltpu.sync_copy(buf1, out_hbm.at[pl.ds(base + W, W)])

    return k(table, idx)


def kernel(x, params):
    p = params
    b, s, _ = x.shape
    h = x + p['posemb']
    h = _attn_block(h, p['block0'])
    h = _mlp_block(h, p['block0'])
    h = _attn_block(h, p['block1'])

    blk = p['block1']
    xg = h.reshape(G, GS, D)
    xln, st, gate, tokslot, aux = _router(xg, blk)

    disp_idx = (st.reshape(G, E, CAP).transpose(1, 0, 2).reshape(E, G * CAP))
    disp_idx = jnp.pad(disp_idx, ((0, 0), (0, SLOT - G * CAP))).reshape(NSLOT)

    buf = _sc_gather(xln.reshape(T, D), disp_idx)
    ebuf = _expert_ffn(buf.reshape(E, SLOT, D), blk)
    yraw = _sc_gather(ebuf.reshape(NSLOT, D), tokslot.reshape(T))

    out = _final(h.reshape(T, D), yraw, gate.reshape(T, 1),
                 p['lnf_s'], p['lnf_b'])
    return out.reshape(b, s, D), jnp.sum(aux)
